# baseline (speedup 1.0000x reference)
import jax
import jax.numpy as jnp
from jax.experimental import pallas as pl
from jax.experimental.pallas import tpu as pltpu

B = 8192
D = 768
L = 256
K = 8192

BB = 1024
KC = 4096
LANES = 128


def _prep_body(cc_ref, ccb_ref, c2_ref):
    cf = cc_ref[...]
    c2 = jnp.sum(cf * cf, axis=1)
    c2b = (0.5 * c2).reshape(1, K).astype(jnp.bfloat16)
    c2_ref[...] = jnp.broadcast_to(c2b, (16, K))
    ccb_ref[...] = cf.astype(jnp.bfloat16)


def _loss_body(x_ref, a_ref, h_ref, ccb_ref, c2_ref, out_ref):
    i = pl.program_id(0)

    diff = x_ref[...] - a_ref[...]
    recon = jnp.sum(diff * diff)

    h = h_ref[...]
    h2 = jnp.sum(h * h, axis=1)
    hb = h.astype(jnp.bfloat16)

    m = jnp.full((BB, LANES), -jnp.inf, dtype=jnp.bfloat16)
    for kc in range(K // KC):
        s = jax.lax.dot_general(
            hb, ccb_ref[kc * KC:(kc + 1) * KC, :],
            (((1,), (1,)), ((), ())),
            preferred_element_type=jnp.float32,
        )
        sb = s.astype(jnp.bfloat16) - c2_ref[0:1, kc * KC:(kc + 1) * KC]
        for t in range(KC // LANES):
            m = jnp.maximum(m, sb[:, t * LANES:(t + 1) * LANES])
    m_row = jnp.max(m.astype(jnp.float32), axis=1)

    d2 = h2 - 2.0 * m_row
    part = jnp.reshape(recon / (B * D) + jnp.sum(d2) / (B * L), (1, 1))

    @pl.when(i == 0)
    def _():
        out_ref[...] = jnp.zeros((1, 1), jnp.float32)
    out_ref[...] += part


def kernel(x, h_x, a_x, cluster_centers):
    ccb, c2 = pl.pallas_call(
        _prep_body,
        out_shape=[jax.ShapeDtypeStruct((K, L), jnp.bfloat16),
                   jax.ShapeDtypeStruct((16, K), jnp.bfloat16)],
    )(cluster_centers)
    out = pl.pallas_call(
        _loss_body,
        grid=(B // BB,),
        in_specs=[
            pl.BlockSpec((BB, D), lambda i: (i, 0)),
            pl.BlockSpec((BB, D), lambda i: (i, 0)),
            pl.BlockSpec((BB, L), lambda i: (i, 0)),
            pl.BlockSpec((K, L), lambda i: (0, 0)),
            pl.BlockSpec((16, K), lambda i: (0, 0)),
        ],
        out_specs=pl.BlockSpec((1, 1), lambda i: (0, 0)),
        out_shape=jax.ShapeDtypeStruct((1, 1), jnp.float32),
    )(x, a_x, h_x, ccb, c2)
    return out[0, 0]
